# baseline (speedup 1.0000x reference)
import jax
import jax.numpy as jnp
from jax.experimental import pallas as pl
from jax.experimental.pallas import tpu as pltpu


def _gnn_kernel(f_ref, w_ref, adj_ref, o_ref, s_ref):
    @pl.when(pl.program_id(0) == 0)
    def _():
        s_ref[...] = jnp.dot(
            f_ref[...], w_ref[...], preferred_element_type=jnp.float32
        )

    o_ref[...] = jnp.maximum(
        jnp.dot(adj_ref[...], s_ref[...], preferred_element_type=jnp.float32),
        0.0,
    )


def kernel(features, adj, W):
    n, d_in = features.shape
    d_out = W.shape[1]
    bm = 512
    grid = (n // bm,)
    return pl.pallas_call(
        _gnn_kernel,
        grid=grid,
        in_specs=[
            pl.BlockSpec((n, d_in), lambda i: (0, 0)),
            pl.BlockSpec((d_in, d_out), lambda i: (0, 0)),
            pl.BlockSpec((bm, n), lambda i: (i, 0)),
        ],
        out_specs=pl.BlockSpec((bm, d_out), lambda i: (i, 0)),
        out_shape=jax.ShapeDtypeStruct((n, d_out), jnp.float32),
        scratch_shapes=[pltpu.VMEM((n, d_out), jnp.float32)],
    )(features, W, adj)
